# baseline (speedup 1.0000x reference)
import functools

import jax
import jax.numpy as jnp
from jax import lax
from jax.experimental import pallas as pl
from jax.experimental.pallas import tpu as pltpu
from jax.experimental.pallas import tpu_sc as plsc

_N_WAY = 5
_K_SHOT = 5
_NEIGHBOR_K = 3
_B = 8
_Q = 75
_QP = 80
_C = 640
_HW = 25
_QG = _QP // 16
_GW = _HW * 16
_SR = _N_WAY * _K_SHOT * _HW
_PER_CLASS = _K_SHOT * _HW
_PC_PAD = 128
_ITEMS = _B * _N_WAY * _QG
_NSUB = 32
_EPS = 1e-8
_NEG = -1e30


def _tc_sim_body(qv_ref, sv_ref, out_ref):
    qv = qv_ref[0]
    sv = sv_ref[0]
    qn = jnp.sqrt(jnp.sum(qv * qv, axis=1, keepdims=True)) + _EPS
    qv = qv / qn
    sn = jnp.sqrt(jnp.sum(sv * sv, axis=1, keepdims=True)) + _EPS
    sv = sv / sn
    qv = qv.astype(jnp.bfloat16)
    sv = sv.astype(jnp.bfloat16)
    qv = jnp.concatenate(
        [qv, jnp.zeros((_QG * _GW - _Q * _HW, _C), jnp.bfloat16)], axis=0)
    pr = lax.broadcasted_iota(jnp.int32, (_GW, _GW), 0)
    pc = lax.broadcasted_iota(jnp.int32, (_GW, _GW), 1)
    perm = jnp.where(pr == (pc % 16) * _HW + pc // 16, 1.0, 0.0)
    perm = perm.astype(jnp.bfloat16)
    for n in range(_N_WAY):
        svn = sv[n * _PER_CLASS:(n + 1) * _PER_CLASS, :]
        for qg in range(_QG):
            qvg = qv[qg * _GW:(qg + 1) * _GW, :]
            sim = lax.dot_general(svn, qvg, (((1,), (1,)), ((), ())),
                                  preferred_element_type=jnp.float32)
            sim = lax.dot_general(sim.astype(jnp.bfloat16), perm,
                                  (((1,), (0,)), ((), ())),
                                  preferred_element_type=jnp.float32)
            sim = jnp.concatenate(
                [sim, jnp.full((_PC_PAD - _PER_CLASS, _GW), _NEG,
                               jnp.float32)], axis=0)
            out_ref[0, n, qg] = sim


_QROWS = (32, 32, 32, 32)
_QOFF = (0, 32, 64, 96)
_NQ = 4


def _sc_top3_body(items, sim_hbm, out_hbm, b0, b1, b2, b3, tstate, accbuf,
                  s0, s1, s2, s3):
    wid = lax.axis_index("s") * 2 + lax.axis_index("c")
    bufs = (b0, b1, b2, b3)
    sems = (s0, s1, s2, s3)
    n_k = (items + _NSUB - 1) // _NSUB
    full_k = items // _NSUB
    tail_w = items - full_k * _NSUB
    n_h = n_k * _NQ

    def coords(k):
        t = k * _NSUB + wid
        return (t, t // (_N_WAY * _QG), (t // _QG) % _N_WAY, t % _QG)

    def quarter_copy(h):
        k, qi = h // _NQ, h % _NQ
        _, b, n, qg = coords(k)
        cnt = _QROWS[qi]
        src = sim_hbm.at[b, n, qg, pl.ds(_QOFF[qi], cnt), :]
        dst = bufs[h % _NQ].at[pl.ds(0, cnt), :]
        return pltpu.make_async_copy(src, dst, sems[h % _NQ])

    def compute_quarter(h):
        k, qi = h // _NQ, h % _NQ
        t = k * _NSUB + wid
        buf = bufs[h % _NQ]
        cnt = _QROWS[qi]

        def ibody(i, acc):
            col = pl.multiple_of(i * 16, 16)
            if qi == 0:
                init = (jnp.full((16,), _NEG, jnp.float32),
                        jnp.full((16,), _NEG, jnp.float32),
                        jnp.full((16,), _NEG, jnp.float32))
            else:
                init = (tstate[i, 0], tstate[i, 1], tstate[i, 2])

            def jbody(j, carry):
                t1, t2, t3 = carry
                v = buf[j, pl.ds(col, 16)]
                m1 = jnp.maximum(t1, v)
                r1 = jnp.minimum(t1, v)
                m2 = jnp.maximum(t2, r1)
                r2 = jnp.minimum(t2, r1)
                m3 = jnp.maximum(t3, r2)
                return (m1, m2, m3)

            t1, t2, t3 = lax.fori_loop(0, cnt, jbody, init, unroll=4)
            if qi + 1 < _NQ:
                tstate[i, 0] = t1
                tstate[i, 1] = t2
                tstate[i, 2] = t3
                return acc
            return acc + (t1 + t2 + t3) * (1.0 / float(_NEIGHBOR_K))

        acc = lax.fori_loop(0, _HW, ibody, jnp.zeros((16,), jnp.float32))
        if qi + 1 == _NQ:
            accbuf[...] = acc
            pltpu.sync_copy(accbuf, out_hbm.at[t])

    def guarded(h, fn):
        if h // _NQ >= full_k:
            @pl.when(wid < tail_w)
            def _():
                fn()
        else:
            fn()

    for h in range(_NQ):
        guarded(h, lambda h=h: quarter_copy(h).start())
    for h in range(n_h):
        guarded(h, lambda h=h: quarter_copy(h).wait())
        if h + _NQ < n_h:
            guarded(h + _NQ, lambda h=h: quarter_copy(h + _NQ).start())
        guarded(h, lambda h=h: compute_quarter(h))


def _tc_loss_body(lg_ref, qy_ref, out_ref):
    lg = lg_ref[...]
    lab = qy_ref[...]
    mx = jnp.max(lg, axis=1, keepdims=True)
    sh = lg - mx
    lse = jnp.log(jnp.sum(jnp.exp(sh), axis=1, keepdims=True))
    logp = sh - lse
    onehot = lax.broadcasted_iota(jnp.int32, (_B * _Q, _N_WAY), 1) == lab
    out_ref[:, :] = -jnp.sum(jnp.where(onehot, logp, 0.0), axis=(0, 1),
                             keepdims=True) / float(_B * _Q)


def kernel(support_xf, support_y, query_xf, query_y):
    del support_y
    b, q, c, h, w = query_xf.shape
    hw = h * w

    qv = query_xf.reshape(b, q, c, hw).transpose(0, 1, 3, 2)
    qv = qv.reshape(b, q * hw, c)
    sv = support_xf.reshape(b, _SR // hw, c, hw).transpose(0, 1, 3, 2)
    sv = sv.reshape(b, _SR, c)

    def run_chunk(qvc, svc, nb):
        sim = pl.pallas_call(
            _tc_sim_body,
            grid=(nb,),
            in_specs=[
                pl.BlockSpec((1, q * hw, c), lambda i: (i, 0, 0)),
                pl.BlockSpec((1, _SR, c), lambda i: (i, 0, 0)),
            ],
            out_specs=pl.BlockSpec((1, _N_WAY, _QG, _PC_PAD, _GW),
                                   lambda i: (i, 0, 0, 0, 0)),
            out_shape=jax.ShapeDtypeStruct((nb, _N_WAY, _QG, _PC_PAD, _GW),
                                           jnp.float32),
        )(qvc, svc)

        items = nb * _N_WAY * _QG
        sc_top3 = pl.kernel(
            functools.partial(_sc_top3_body, items),
            out_type=jax.ShapeDtypeStruct((items, 16), jnp.float32),
            mesh=plsc.VectorSubcoreMesh(core_axis_name="c",
                                        subcore_axis_name="s",
                                        num_cores=2, num_subcores=16),
            scratch_types=[
                pltpu.VMEM((_QROWS[0], _GW), jnp.float32),
                pltpu.VMEM((_QROWS[0], _GW), jnp.float32),
                pltpu.VMEM((_QROWS[0], _GW), jnp.float32),
                pltpu.VMEM((_QROWS[0], _GW), jnp.float32),
                pltpu.VMEM((_HW, 3, 16), jnp.float32),
                pltpu.VMEM((16,), jnp.float32),
                pltpu.SemaphoreType.DMA,
                pltpu.SemaphoreType.DMA,
                pltpu.SemaphoreType.DMA,
                pltpu.SemaphoreType.DMA,
            ],
        )
        return sc_top3(sim)

    half = b // 2
    cls0 = run_chunk(qv[:half], sv[:half], half)
    cls1 = run_chunk(qv[half:], sv[half:], b - half)
    cls = jnp.concatenate([cls0, cls1], axis=0)

    logits = cls.reshape(b, _N_WAY, _QP)[:, :, :q].transpose(0, 2, 1)
    lg = logits.reshape(b * q, _N_WAY)
    qy = query_y.reshape(b * q, 1).astype(jnp.int32)

    loss = pl.pallas_call(
        _tc_loss_body,
        out_shape=jax.ShapeDtypeStruct((1, 1), jnp.float32),
    )(lg, qy)
    return loss[0, 0]


# speedup vs baseline: 1.2935x; 1.2935x over previous
import functools

import jax
import jax.numpy as jnp
from jax import lax
from jax.experimental import pallas as pl
from jax.experimental.pallas import tpu as pltpu
from jax.experimental.pallas import tpu_sc as plsc

_N_WAY = 5
_K_SHOT = 5
_NEIGHBOR_K = 3
_B = 8
_Q = 75
_QP = 80
_C = 640
_HW = 25
_QG = _QP // 16
_GW = _HW * 16
_SR = _N_WAY * _K_SHOT * _HW
_PER_CLASS = _K_SHOT * _HW
_PC_PAD = 128
_ITEMS = _B * _N_WAY * _QG
_NSUB = 32
_EPS = 1e-8
_NEG = -1e30


def _tc_sim_body(qv_ref, sv_ref, out_ref):
    qv = qv_ref[0].astype(jnp.float32)
    sv = sv_ref[0].astype(jnp.float32)
    qn = jnp.sqrt(jnp.sum(qv * qv, axis=1, keepdims=True)) + _EPS
    qv = (qv / qn).astype(jnp.bfloat16)
    sn = jnp.sqrt(jnp.sum(sv * sv, axis=1, keepdims=True)) + _EPS
    sv = (sv / sn).astype(jnp.bfloat16)
    qv = jnp.concatenate(
        [qv, jnp.zeros((_QG * _GW - _Q * _HW, _C), jnp.bfloat16)], axis=0)
    pr = lax.broadcasted_iota(jnp.int32, (_GW, _GW), 0)
    pc = lax.broadcasted_iota(jnp.int32, (_GW, _GW), 1)
    perm = jnp.where(pr == (pc % 16) * _HW + pc // 16, 1.0, 0.0)
    perm = perm.astype(jnp.bfloat16)
    for n in range(_N_WAY):
        svn = sv[n * _PER_CLASS:(n + 1) * _PER_CLASS, :]
        for qg in range(_QG):
            qvg = qv[qg * _GW:(qg + 1) * _GW, :]
            sim = lax.dot_general(svn, qvg, (((1,), (1,)), ((), ())),
                                  preferred_element_type=jnp.float32)
            sim = lax.dot_general(sim.astype(jnp.bfloat16), perm,
                                  (((1,), (0,)), ((), ())),
                                  preferred_element_type=jnp.float32)
            sim = jnp.concatenate(
                [sim, jnp.full((_PC_PAD - _PER_CLASS, _GW), _NEG,
                               jnp.float32)], axis=0)
            out_ref[0, n, qg] = sim


_QROWS = (32, 32, 32, 32)
_QOFF = (0, 32, 64, 96)
_NQ = 4


def _sc_top3_body(items, sim_hbm, out_hbm, b0, b1, b2, b3, tstate, accbuf,
                  s0, s1, s2, s3):
    wid = lax.axis_index("s") * 2 + lax.axis_index("c")
    bufs = (b0, b1, b2, b3)
    sems = (s0, s1, s2, s3)
    n_k = (items + _NSUB - 1) // _NSUB
    full_k = items // _NSUB
    tail_w = items - full_k * _NSUB
    n_h = n_k * _NQ

    def coords(k):
        t = k * _NSUB + wid
        return (t, t // (_N_WAY * _QG), (t // _QG) % _N_WAY, t % _QG)

    def quarter_copy(h):
        k, qi = h // _NQ, h % _NQ
        _, b, n, qg = coords(k)
        cnt = _QROWS[qi]
        src = sim_hbm.at[b, n, qg, pl.ds(_QOFF[qi], cnt), :]
        dst = bufs[h % _NQ].at[pl.ds(0, cnt), :]
        return pltpu.make_async_copy(src, dst, sems[h % _NQ])

    def compute_quarter(h):
        k, qi = h // _NQ, h % _NQ
        t = k * _NSUB + wid
        buf = bufs[h % _NQ]
        cnt = _QROWS[qi]

        def ibody(i, acc):
            col = pl.multiple_of(i * 16, 16)
            if qi == 0:
                init = (jnp.full((16,), _NEG, jnp.float32),
                        jnp.full((16,), _NEG, jnp.float32),
                        jnp.full((16,), _NEG, jnp.float32))
            else:
                init = (tstate[i, 0], tstate[i, 1], tstate[i, 2])

            def jbody(j, carry):
                t1, t2, t3 = carry
                v = buf[j, pl.ds(col, 16)]
                m1 = jnp.maximum(t1, v)
                r1 = jnp.minimum(t1, v)
                m2 = jnp.maximum(t2, r1)
                r2 = jnp.minimum(t2, r1)
                m3 = jnp.maximum(t3, r2)
                return (m1, m2, m3)

            t1, t2, t3 = lax.fori_loop(0, cnt, jbody, init, unroll=4)
            if qi + 1 < _NQ:
                tstate[i, 0] = t1
                tstate[i, 1] = t2
                tstate[i, 2] = t3
                return acc
            return acc + (t1 + t2 + t3) * (1.0 / float(_NEIGHBOR_K))

        acc = lax.fori_loop(0, _HW, ibody, jnp.zeros((16,), jnp.float32))
        if qi + 1 == _NQ:
            accbuf[...] = acc
            pltpu.sync_copy(accbuf, out_hbm.at[t])

    def guarded(h, fn):
        if h // _NQ >= full_k:
            @pl.when(wid < tail_w)
            def _():
                fn()
        else:
            fn()

    for h in range(_NQ):
        guarded(h, lambda h=h: quarter_copy(h).start())
    for h in range(n_h):
        guarded(h, lambda h=h: quarter_copy(h).wait())
        if h + _NQ < n_h:
            guarded(h + _NQ, lambda h=h: quarter_copy(h + _NQ).start())
        guarded(h, lambda h=h: compute_quarter(h))


def _tc_loss_body(lg_ref, qy_ref, out_ref):
    lg = lg_ref[...]
    lab = qy_ref[...]
    mx = jnp.max(lg, axis=1, keepdims=True)
    sh = lg - mx
    lse = jnp.log(jnp.sum(jnp.exp(sh), axis=1, keepdims=True))
    logp = sh - lse
    onehot = lax.broadcasted_iota(jnp.int32, (_B * _Q, _N_WAY), 1) == lab
    out_ref[:, :] = -jnp.sum(jnp.where(onehot, logp, 0.0), axis=(0, 1),
                             keepdims=True) / float(_B * _Q)


def kernel(support_xf, support_y, query_xf, query_y):
    del support_y
    b, q, c, h, w = query_xf.shape
    hw = h * w

    qv = query_xf.astype(jnp.bfloat16)
    qv = qv.reshape(b, q, c, hw).transpose(0, 1, 3, 2)
    qv = qv.reshape(b, q * hw, c)
    sv = support_xf.astype(jnp.bfloat16)
    sv = sv.reshape(b, _SR // hw, c, hw).transpose(0, 1, 3, 2)
    sv = sv.reshape(b, _SR, c)

    def run_chunk(qvc, svc, nb):
        sim = pl.pallas_call(
            _tc_sim_body,
            grid=(nb,),
            in_specs=[
                pl.BlockSpec((1, q * hw, c), lambda i: (i, 0, 0)),
                pl.BlockSpec((1, _SR, c), lambda i: (i, 0, 0)),
            ],
            out_specs=pl.BlockSpec((1, _N_WAY, _QG, _PC_PAD, _GW),
                                   lambda i: (i, 0, 0, 0, 0)),
            out_shape=jax.ShapeDtypeStruct((nb, _N_WAY, _QG, _PC_PAD, _GW),
                                           jnp.float32),
        )(qvc, svc)

        items = nb * _N_WAY * _QG
        sc_top3 = pl.kernel(
            functools.partial(_sc_top3_body, items),
            out_type=jax.ShapeDtypeStruct((items, 16), jnp.float32),
            mesh=plsc.VectorSubcoreMesh(core_axis_name="c",
                                        subcore_axis_name="s",
                                        num_cores=2, num_subcores=16),
            scratch_types=[
                pltpu.VMEM((_QROWS[0], _GW), jnp.float32),
                pltpu.VMEM((_QROWS[0], _GW), jnp.float32),
                pltpu.VMEM((_QROWS[0], _GW), jnp.float32),
                pltpu.VMEM((_QROWS[0], _GW), jnp.float32),
                pltpu.VMEM((_HW, 3, 16), jnp.float32),
                pltpu.VMEM((16,), jnp.float32),
                pltpu.SemaphoreType.DMA,
                pltpu.SemaphoreType.DMA,
                pltpu.SemaphoreType.DMA,
                pltpu.SemaphoreType.DMA,
            ],
        )
        return sc_top3(sim)

    cls = run_chunk(qv, sv, b)

    logits = cls.reshape(b, _N_WAY, _QP)[:, :, :q].transpose(0, 2, 1)
    lg = logits.reshape(b * q, _N_WAY)
    qy = query_y.reshape(b * q, 1).astype(jnp.int32)

    loss = pl.pallas_call(
        _tc_loss_body,
        out_shape=jax.ShapeDtypeStruct((1, 1), jnp.float32),
    )(lg, qy)
    return loss[0, 0]
